# Initial kernel scaffold; baseline (speedup 1.0000x reference)
import jax
import jax.numpy as jnp
from jax.experimental import pallas as pl


def kernel(indices_single, indices_multi, weights_multi, table):
    raise NotImplementedError("write your pallas kernel here")


# speedup vs baseline: 2.0306x; 2.0306x over previous
import functools

import jax
import jax.numpy as jnp
from jax import lax
from jax.experimental import pallas as pl
from jax.experimental.pallas import tpu as pltpu
from jax.experimental.pallas import tpu_sc as plsc

B, L, D = 16384, 50, 32
NC, NS = 2, 16
NW = NC * NS
BPW = B // NW
C = 32
NCHUNK = BPW // C
CL = C * L
NIDX = 13 * 128


def _emb_kernel(idx_m_hbm, idx_s_hbm, w_hbm, table_hbm, out_hbm,
                idxm_v, idxs_v, w_v, rowsm_v, rowss_v, out_v, sem_m, sem_s):
    wid = lax.axis_index("s") * NC + lax.axis_index("c")

    zi = jnp.zeros((16,), jnp.int32)
    for k in range((NIDX - CL) // 16):
        idxm_v[pl.ds(CL + 16 * k, 16)] = zi

    def chunk_body(g, _):
        base = wid * BPW + g * C
        pltpu.sync_copy(idx_s_hbm.at[pl.ds(base, C)], idxs_v)
        pltpu.sync_copy(idx_m_hbm.at[pl.ds(base * L, CL)],
                        idxm_v.at[pl.ds(0, CL)])
        pltpu.sync_copy(w_hbm.at[pl.ds(base * L, CL)], w_v)
        cps = [pltpu.async_copy(table_hbm.at[idxm_v.at[pl.ds(j * 128, 128)]],
                                rowsm_v.at[pl.ds(j * 128, 128)], sem_m)
               for j in range(NIDX // 128)]
        cp_s = pltpu.async_copy(table_hbm.at[idxs_v], rowss_v, sem_s)
        for cp in cps:
            cp.wait()
        cp_s.wait()

        def row_body(c, _):
            rowbase = c * L

            def l_body(l, carry):
                acc0, acc1, wsum = carry
                r = rowbase + l
                wv = plsc.load_gather(w_v, [jnp.full((16,), r, jnp.int32)])
                acc0 = acc0 + wv * rowsm_v[r, pl.ds(0, 16)]
                acc1 = acc1 + wv * rowsm_v[r, pl.ds(16, 16)]
                return acc0, acc1, wsum + wv

            z = jnp.zeros((16,), jnp.float32)
            acc0, acc1, wsum = lax.fori_loop(0, L, l_body, (z, z, z))
            out_v[c, pl.ds(0, 16)] = rowss_v[c, pl.ds(0, 16)]
            out_v[c, pl.ds(16, 16)] = rowss_v[c, pl.ds(16, 16)]
            out_v[c, pl.ds(32, 16)] = acc0 / wsum
            out_v[c, pl.ds(48, 16)] = acc1 / wsum
            return 0

        lax.fori_loop(0, C, row_body, 0)
        pltpu.sync_copy(out_v, out_hbm.at[pl.ds(base, C)])
        return 0

    lax.fori_loop(0, NCHUNK, chunk_body, 0)


def kernel(indices_single, indices_multi, weights_multi, table):
    idx_m = indices_multi.reshape(-1).astype(jnp.int32)
    idx_s = indices_single.astype(jnp.int32)
    w = weights_multi.reshape(-1)
    mesh = plsc.VectorSubcoreMesh(core_axis_name="c", subcore_axis_name="s",
                                  num_cores=NC, num_subcores=NS)
    run = functools.partial(
        pl.kernel, mesh=mesh,
        compiler_params=pltpu.CompilerParams(needs_layout_passes=False,
                                             use_tc_tiling_on_sc=False),
        out_type=jax.ShapeDtypeStruct((B, 2 * D), jnp.float32),
        scratch_types=[
            pltpu.VMEM((NIDX,), jnp.int32),
            pltpu.VMEM((C,), jnp.int32),
            pltpu.VMEM((CL,), jnp.float32),
            pltpu.VMEM((NIDX, D), jnp.float32),
            pltpu.VMEM((C, D), jnp.float32),
            pltpu.VMEM((C, 2 * D), jnp.float32),
            pltpu.SemaphoreType.DMA,
            pltpu.SemaphoreType.DMA,
        ],
    )(_emb_kernel)
    return run(idx_m, idx_s, w, table)


# speedup vs baseline: 2.0386x; 1.0040x over previous
import functools

import jax
import jax.numpy as jnp
from jax import lax
from jax.experimental import pallas as pl
from jax.experimental.pallas import tpu as pltpu
from jax.experimental.pallas import tpu_sc as plsc

B, L, D = 16384, 50, 32
NC, NS = 2, 16
NW = NC * NS
BPW = B // NW
C = 32
NCHUNK = BPW // C
CL = C * L
NIDX = 13 * 128


def _emb_kernel(idx_m_hbm, idx_s_hbm, w_hbm, table_hbm, out_hbm,
                idxm_v, idxs_v, w_v, rowsm_v, rowss_v, out_v, sem_m, sem_s):
    wid = lax.axis_index("s") * NC + lax.axis_index("c")

    zi = jnp.zeros((16,), jnp.int32)
    for k in range((NIDX - CL) // 16):
        idxm_v[pl.ds(CL + 16 * k, 16)] = zi

    def chunk_body(g, _):
        base = wid * BPW + g * C
        pltpu.sync_copy(idx_s_hbm.at[pl.ds(base, C)], idxs_v)
        pltpu.sync_copy(idx_m_hbm.at[pl.ds(base * L, CL)],
                        idxm_v.at[pl.ds(0, CL)])
        pltpu.sync_copy(w_hbm.at[pl.ds(base * L, CL)], w_v)
        cps = [pltpu.async_copy(table_hbm.at[idxm_v.at[pl.ds(j * 128, 128)]],
                                rowsm_v.at[pl.ds(j * 128, 128)], sem_m)
               for j in range(NIDX // 128)]
        cp_s = pltpu.async_copy(table_hbm.at[idxs_v], rowss_v, sem_s)
        for cp in cps:
            cp.wait()
        cp_s.wait()

        def row_body(c, _):
            rowbase = c * L
            rbase = jnp.full((16,), rowbase, jnp.int32)

            z = jnp.zeros((16,), jnp.float32)
            a0 = [z, z]
            a1 = [z, z]
            ws = [z, z]
            for l in range(L):
                p = l & 1
                r = rowbase + l
                wv = plsc.load_gather(w_v, [rbase + l])
                a0[p] = a0[p] + wv * rowsm_v[r, pl.ds(0, 16)]
                a1[p] = a1[p] + wv * rowsm_v[r, pl.ds(16, 16)]
                ws[p] = ws[p] + wv
            acc0 = a0[0] + a0[1]
            acc1 = a1[0] + a1[1]
            wsum = ws[0] + ws[1]
            out_v[c, pl.ds(0, 16)] = rowss_v[c, pl.ds(0, 16)]
            out_v[c, pl.ds(16, 16)] = rowss_v[c, pl.ds(16, 16)]
            out_v[c, pl.ds(32, 16)] = acc0 / wsum
            out_v[c, pl.ds(48, 16)] = acc1 / wsum
            return 0

        lax.fori_loop(0, C, row_body, 0)
        pltpu.sync_copy(out_v, out_hbm.at[pl.ds(base, C)])
        return 0

    lax.fori_loop(0, NCHUNK, chunk_body, 0)


def kernel(indices_single, indices_multi, weights_multi, table):
    idx_m = indices_multi.reshape(-1).astype(jnp.int32)
    idx_s = indices_single.astype(jnp.int32)
    w = weights_multi.reshape(-1)
    mesh = plsc.VectorSubcoreMesh(core_axis_name="c", subcore_axis_name="s",
                                  num_cores=NC, num_subcores=NS)
    run = functools.partial(
        pl.kernel, mesh=mesh,
        compiler_params=pltpu.CompilerParams(needs_layout_passes=False,
                                             use_tc_tiling_on_sc=False),
        out_type=jax.ShapeDtypeStruct((B, 2 * D), jnp.float32),
        scratch_types=[
            pltpu.VMEM((NIDX,), jnp.int32),
            pltpu.VMEM((C,), jnp.int32),
            pltpu.VMEM((CL,), jnp.float32),
            pltpu.VMEM((NIDX, D), jnp.float32),
            pltpu.VMEM((C, D), jnp.float32),
            pltpu.VMEM((C, 2 * D), jnp.float32),
            pltpu.SemaphoreType.DMA,
            pltpu.SemaphoreType.DMA,
        ],
    )(_emb_kernel)
    return run(idx_m, idx_s, w, table)


# speedup vs baseline: 2.9665x; 1.4551x over previous
import functools

import jax
import jax.numpy as jnp
from jax import lax
from jax.experimental import pallas as pl
from jax.experimental.pallas import tpu as pltpu
from jax.experimental.pallas import tpu_sc as plsc

B, L, D = 16384, 50, 32
NC, NS = 2, 16
NW = NC * NS
BPW = B // NW
C = 64
NCHUNK = BPW // C
CL = C * L
NSTREAM = CL // 128


def _emb_kernel(idx_m_hbm, idx_s_hbm, w_hbm, table_hbm, out_hbm,
                idxm_v, idxs_v, w_v, rowsm_v, rowss_v, out_v, sem_i, sem_m):
    wid = lax.axis_index("s") * NC + lax.axis_index("c")

    def chunk_body(g, _):
        base = wid * BPW + g * C
        cp_i = pltpu.async_copy(idx_m_hbm.at[pl.ds(base * L, CL)], idxm_v,
                                sem_i)
        cp_s = pltpu.async_copy(idx_s_hbm.at[pl.ds(base, C)], idxs_v, sem_i)
        cp_w = pltpu.async_copy(w_hbm.at[pl.ds(base * L, CL)], w_v, sem_i)
        cp_i.wait()
        cp_s.wait()
        cps = [pltpu.async_copy(table_hbm.at[idxm_v.at[pl.ds(j * 128, 128)]],
                                rowsm_v.at[pl.ds(j * 128, 128)], sem_m)
               for j in range(NSTREAM)]
        cp_r = pltpu.async_copy(table_hbm.at[idxs_v], rowss_v, sem_m)
        cp_w.wait()
        for cp in cps:
            cp.wait()
        cp_r.wait()

        def row_body(c, _):
            rowbase = c * L
            rbase = jnp.full((16,), rowbase, jnp.int32)

            z = jnp.zeros((16,), jnp.float32)
            a0 = [z, z]
            a1 = [z, z]
            ws = [z, z]
            for l in range(L):
                p = l & 1
                r = rowbase + l
                wv = plsc.load_gather(w_v, [rbase + l])
                a0[p] = a0[p] + wv * rowsm_v[r, pl.ds(0, 16)]
                a1[p] = a1[p] + wv * rowsm_v[r, pl.ds(16, 16)]
                ws[p] = ws[p] + wv
            acc0 = a0[0] + a0[1]
            acc1 = a1[0] + a1[1]
            wsum = ws[0] + ws[1]
            out_v[c, pl.ds(0, 16)] = rowss_v[c, pl.ds(0, 16)]
            out_v[c, pl.ds(16, 16)] = rowss_v[c, pl.ds(16, 16)]
            out_v[c, pl.ds(32, 16)] = acc0 / wsum
            out_v[c, pl.ds(48, 16)] = acc1 / wsum
            return 0

        lax.fori_loop(0, C, row_body, 0)
        pltpu.sync_copy(out_v, out_hbm.at[pl.ds(base, C)])
        return 0

    lax.fori_loop(0, NCHUNK, chunk_body, 0)


def kernel(indices_single, indices_multi, weights_multi, table):
    idx_m = indices_multi.reshape(-1).astype(jnp.int32)
    idx_s = indices_single.astype(jnp.int32)
    w = weights_multi.reshape(-1)
    mesh = plsc.VectorSubcoreMesh(core_axis_name="c", subcore_axis_name="s",
                                  num_cores=NC, num_subcores=NS)
    run = functools.partial(
        pl.kernel, mesh=mesh,
        compiler_params=pltpu.CompilerParams(needs_layout_passes=False,
                                             use_tc_tiling_on_sc=False),
        out_type=jax.ShapeDtypeStruct((B, 2 * D), jnp.float32),
        scratch_types=[
            pltpu.VMEM((CL,), jnp.int32),
            pltpu.VMEM((C,), jnp.int32),
            pltpu.VMEM((CL,), jnp.float32),
            pltpu.VMEM((CL, D), jnp.float32),
            pltpu.VMEM((C, D), jnp.float32),
            pltpu.VMEM((C, 2 * D), jnp.float32),
            pltpu.SemaphoreType.DMA,
            pltpu.SemaphoreType.DMA,
        ],
    )(_emb_kernel)
    return run(idx_m, idx_s, w, table)


# speedup vs baseline: 3.1629x; 1.0662x over previous
import functools

import jax
import jax.numpy as jnp
from jax import lax
from jax.experimental import pallas as pl
from jax.experimental.pallas import tpu as pltpu
from jax.experimental.pallas import tpu_sc as plsc

B, L, D = 16384, 50, 32
NC, NS = 2, 16
NW = NC * NS
BPW = B // NW
WL = BPW * L
C = 16
NCHUNK = BPW // C
CL = C * L
SL = [128] * 6 + [32]
SOFF = [0, 128, 256, 384, 512, 640, 768]


def _emb_kernel(idx_m_hbm, idx_s_hbm, w_hbm, table_hbm, out_hbm,
                idxm_v, idxs_v, w_v, rowsm_v, rowss_v, out_v,
                sem_g0, sem_g1, sem_o0, sem_o1, sem_st):
    wid = lax.axis_index("s") * NC + lax.axis_index("c")
    wbase = wid * BPW
    sem_g = [sem_g0, sem_g1]
    sem_o = [sem_o0, sem_o1]

    cp1 = pltpu.async_copy(idx_m_hbm.at[pl.ds(wbase * L, WL)], idxm_v, sem_st)
    cp2 = pltpu.async_copy(idx_s_hbm.at[pl.ds(wbase, BPW)], idxs_v, sem_st)
    cp3 = pltpu.async_copy(w_hbm.at[pl.ds(wbase * L, WL)], w_v, sem_st)
    cp1.wait()
    cp2.wait()
    cp3.wait()

    def fire(g, par):
        cps = [pltpu.async_copy(
            table_hbm.at[idxm_v.at[pl.ds(g * CL + SOFF[j], SL[j])]],
            rowsm_v.at[par].at[pl.ds(SOFF[j], SL[j])], sem_g[par])
            for j in range(len(SL))]
        cps.append(pltpu.async_copy(table_hbm.at[idxs_v.at[pl.ds(g * C, C)]],
                                    rowss_v.at[par], sem_g[par]))
        return cps

    def drain(g, par):
        for cp in fire_descs(g, par):
            cp.wait()

    def fire_descs(g, par):
        return [pltpu.make_async_copy(
            table_hbm.at[idxm_v.at[pl.ds(g * CL + SOFF[j], SL[j])]],
            rowsm_v.at[par].at[pl.ds(SOFF[j], SL[j])], sem_g[par])
            for j in range(len(SL))] + [
            pltpu.make_async_copy(table_hbm.at[idxs_v.at[pl.ds(g * C, C)]],
                                  rowss_v.at[par], sem_g[par])]

    def compute(g, par):
        rows = rowsm_v.at[par]
        srows = rowss_v.at[par]
        outb = out_v.at[par]

        def row_body(c, _):
            lookbase = g * CL + c * L
            rowbase = c * L
            lbase = jnp.full((16,), lookbase, jnp.int32)
            z = jnp.zeros((16,), jnp.float32)
            a0 = [z, z]
            a1 = [z, z]
            ws = [z, z]
            for l in range(L):
                p = l & 1
                r = rowbase + l
                wv = plsc.load_gather(w_v, [lbase + l])
                a0[p] = a0[p] + wv * rows[r, pl.ds(0, 16)]
                a1[p] = a1[p] + wv * rows[r, pl.ds(16, 16)]
                ws[p] = ws[p] + wv
            acc0 = a0[0] + a0[1]
            acc1 = a1[0] + a1[1]
            wsum = ws[0] + ws[1]
            outb[c, pl.ds(0, 16)] = srows[c, pl.ds(0, 16)]
            outb[c, pl.ds(16, 16)] = srows[c, pl.ds(16, 16)]
            outb[c, pl.ds(32, 16)] = acc0 / wsum
            outb[c, pl.ds(48, 16)] = acc1 / wsum
            return 0

        lax.fori_loop(0, C, row_body, 0)

    fire(0, 0)

    def step_body(s, _):
        for par in (0, 1):
            g = 2 * s + par
            gn = jnp.minimum(g + 1, NCHUNK - 1)
            fire(gn, 1 - par)

            @pl.when(g >= 2)
            def _():
                pltpu.make_async_copy(
                    out_v.at[par],
                    out_hbm.at[pl.ds(wbase + (g - 2) * C, C)],
                    sem_o[par]).wait()

            drain(g, par)
            compute(g, par)
            pltpu.async_copy(out_v.at[par],
                             out_hbm.at[pl.ds(wbase + g * C, C)], sem_o[par])
        return 0

    lax.fori_loop(0, NCHUNK // 2, step_body, 0)

    drain(NCHUNK - 1, 0)
    for par in (0, 1):
        pltpu.make_async_copy(
            out_v.at[par],
            out_hbm.at[pl.ds(wbase + (NCHUNK - 2 + par) * C, C)],
            sem_o[par]).wait()


def kernel(indices_single, indices_multi, weights_multi, table):
    idx_m = indices_multi.reshape(-1).astype(jnp.int32)
    idx_s = indices_single.astype(jnp.int32)
    w = weights_multi.reshape(-1)
    mesh = plsc.VectorSubcoreMesh(core_axis_name="c", subcore_axis_name="s",
                                  num_cores=NC, num_subcores=NS)
    run = functools.partial(
        pl.kernel, mesh=mesh,
        compiler_params=pltpu.CompilerParams(needs_layout_passes=False,
                                             use_tc_tiling_on_sc=False),
        out_type=jax.ShapeDtypeStruct((B, 2 * D), jnp.float32),
        scratch_types=[
            pltpu.VMEM((WL,), jnp.int32),
            pltpu.VMEM((BPW,), jnp.int32),
            pltpu.VMEM((WL,), jnp.float32),
            pltpu.VMEM((2, CL, D), jnp.float32),
            pltpu.VMEM((2, C, D), jnp.float32),
            pltpu.VMEM((2, C, 2 * D), jnp.float32),
            pltpu.SemaphoreType.DMA,
            pltpu.SemaphoreType.DMA,
            pltpu.SemaphoreType.DMA,
            pltpu.SemaphoreType.DMA,
            pltpu.SemaphoreType.DMA,
        ],
    )(_emb_kernel)
    return run(idx_m, idx_s, w, table)
